# baseline (speedup 1.0000x reference)
import functools

import jax
import jax.numpy as jnp
from jax import lax
from jax.experimental import pallas as pl
from jax.experimental.pallas import tpu as pltpu
from jax.experimental.pallas import tpu_sc as plsc

N = 1024
E = 32768
DE = 8
F = 128

_NC = 2
_NS = 16
_NW = _NC * _NS
_SL = 2
_ES = E // _SL
_CHUNK = _ES // _NW
_JBLK = _CHUNK // 128


def _sc_mesh():
    return plsc.VectorSubcoreMesh(core_axis_name="c", subcore_axis_name="s")


@functools.partial(
    pl.kernel,
    mesh=_sc_mesh(),
    out_type=jax.ShapeDtypeStruct((_ES, F), jnp.float32),
    scratch_types=[
        pltpu.VMEM((_JBLK, 128), jnp.int32),
        pltpu.VMEM((_JBLK, 128, F), jnp.float32),
        pltpu.SemaphoreType.DMA,
        pltpu.SemaphoreType.DMA,
    ],
)
def _sc_gather(table_hbm, src_hbm, out_hbm, idx_v, rows_v, gsem, wsem):
    wid = lax.axis_index("s") * _NC + lax.axis_index("c")
    pltpu.sync_copy(src_hbm.at[wid], idx_v)
    gs = [
        pltpu.async_copy(table_hbm.at[idx_v.at[j]], rows_v.at[j], gsem)
        for j in range(_JBLK)
    ]
    wbs = []
    for j in range(_JBLK):
        gs[j].wait()
        wbs.append(
            pltpu.async_copy(
                rows_v.at[j],
                out_hbm.at[pl.ds(wid * _CHUNK + j * 128, 128)],
                wsem,
            )
        )
    for wb in wbs:
        wb.wait()


@functools.partial(
    pl.kernel,
    mesh=_sc_mesh(),
    out_type=jax.ShapeDtypeStruct((_NC, N, F), jnp.float32),
    scratch_types=[
        pltpu.VMEM((_JBLK, 128), jnp.int32),
        pltpu.VMEM((_JBLK, 128, F), jnp.float32),
        pltpu.VMEM_SHARED((N, F), jnp.float32),
        pltpu.SemaphoreType.DMA,
        pltpu.SemaphoreType.DMA,
    ],
)
def _sc_scatter(msg_hbm, dst_hbm, z_hbm, out_hbm, idx_v, msg_v, acc_sh,
                lsem, ssem):
    cid = lax.axis_index("c")
    sid = lax.axis_index("s")
    wid = sid * _NC + cid
    pltpu.sync_copy(dst_hbm.at[wid], idx_v)
    lds = [
        pltpu.async_copy(
            msg_hbm.at[pl.ds(wid * _CHUNK + j * 128, 128)],
            msg_v.at[j],
            lsem,
        )
        for j in range(_JBLK)
    ]

    @pl.when(sid == 0)
    def _():
        pltpu.sync_copy(z_hbm, acc_sh)

    plsc.subcore_barrier()
    scs = []
    for j in range(_JBLK):
        lds[j].wait()
        scs.append(
            pltpu.async_copy(
                msg_v.at[j], acc_sh.at[idx_v.at[j]], ssem, add=True
            )
        )
    for sc in scs:
        sc.wait()
    plsc.subcore_barrier()

    @pl.when(sid == 0)
    def _():
        pltpu.sync_copy(acc_sh, out_hbm.at[cid])


_EB = 2048


def _make_edge(ic, oc, with_ones):
    K = ic * oc

    def body(ht_ref, src_ref, eat_ref, mt_ref, out_ref):
        wT = lax.dot_general(
            mt_ref[...], eat_ref[...], (((1,), (0,)), ((), ())),
            preferred_element_type=jnp.float32,
        )
        n_iota = lax.broadcasted_iota(jnp.int32, (N, _EB), 0)
        mask = jnp.where(
            n_iota == src_ref[...], 1.0, 0.0
        ).astype(jnp.bfloat16)
        xsT = lax.dot_general(
            ht_ref[...], mask, (((1,), (0,)), ((), ())),
            preferred_element_type=jnp.float32,
        )
        acc = xsT[0:1, :] * jnp.maximum(wT[0:oc, :], 0.0)
        for i in range(1, ic):
            acc = acc + xsT[i:i + 1, :] * jnp.maximum(
                wT[i * oc:(i + 1) * oc, :], 0.0)
        msg = lax.transpose(acc, (1, 0))
        pad = jnp.zeros((_EB, F - oc - 1), jnp.float32)
        marker = jnp.full((_EB, 1), 1.0 if with_ones else 0.0, jnp.float32)
        out_ref[...] = jnp.concatenate([msg, marker, pad], axis=1)

    return pl.pallas_call(
        body,
        grid=(_ES // _EB,),
        in_specs=[
            pl.BlockSpec((F, N), lambda e: (0, 0)),
            pl.BlockSpec((1, _EB), lambda e: (0, e)),
            pl.BlockSpec((16, _EB), lambda e: (0, e)),
            pl.BlockSpec((K, 16), lambda e: (0, 0)),
        ],
        out_specs=pl.BlockSpec((_EB, F), lambda e: (e, 0)),
        out_shape=jax.ShapeDtypeStruct((_ES, F), jnp.float32),
    )


def _make_finalize(ic, oc):
    BN = 128

    def body(*refs):
        p_refs = refs[0:_SL]
        c_refs = refs[_SL:2 * _SL]
        xt_ref, r_ref, b_ref, out_ref = refs[2 * _SL:]
        s = p_refs[0][0] + p_refs[0][1]
        for pr in p_refs[1:]:
            s = s + pr[0] + pr[1]
        cnt = c_refs[0][0][:, 64:65] + c_refs[0][1][:, 64:65]
        for cr in c_refs[1:]:
            cnt = cnt + cr[0][:, 64:65] + cr[1][:, 64:65]
        inv = 1.0 / jnp.maximum(cnt, 1.0)
        xb = lax.transpose(xt_ref[...], (1, 0))
        xr = lax.dot_general(
            xb[:, 0:ic], r_ref[...], (((1,), (0,)), ((), ())),
            preferred_element_type=jnp.float32,
        )
        h = jnp.maximum(s[:, 0:oc] * inv + xr + b_ref[...], 0.0)
        pad = jnp.zeros((BN, F - oc), jnp.float32)
        out_ref[...] = lax.transpose(
            jnp.concatenate([h, pad], axis=1), (1, 0))

    part_spec = pl.BlockSpec((_NC, BN, F), lambda i: (0, i, 0))
    return pl.pallas_call(
        body,
        grid=(N // BN,),
        in_specs=(
            [part_spec] * _SL
            + [part_spec] * _SL
            + [
                pl.BlockSpec((F, BN), lambda i: (0, i)),
                pl.BlockSpec((ic, oc), lambda i: (0, 0)),
                pl.BlockSpec((1, oc), lambda i: (0, 0)),
            ]
        ),
        out_specs=pl.BlockSpec((F, BN), lambda i: (0, i)),
        out_shape=jax.ShapeDtypeStruct((F, N), jnp.float32),
    )


def _make_cbt(dh):
    BT = 128

    def body(hti_ref, htj_ref, out_ref):
        hi = lax.transpose(hti_ref[...], (1, 0))
        ht = htj_ref[...]
        acc = jnp.abs(hi[:, 0:1] - ht[0:1, :])
        for kk in range(1, dh):
            acc = acc + jnp.abs(hi[:, kk:kk + 1] - ht[kk:kk + 1, :])
        out_ref[...] = acc

    return pl.pallas_call(
        body,
        grid=(N // BT, N // BT),
        in_specs=[
            pl.BlockSpec((dh, BT), lambda i, j: (0, i)),
            pl.BlockSpec((dh, BT), lambda i, j: (0, j)),
        ],
        out_specs=pl.BlockSpec((BT, BT), lambda i, j: (i, j)),
        out_shape=jax.ShapeDtypeStruct((N, N), jnp.float32),
    )


def _layer(ht, src_sl, ea_sl, dst_sl, zF, ic, oc, with_ones, combo):
    edge = _make_edge(ic, oc, with_ones)
    ht_bf = ht.astype(jnp.bfloat16)
    ms = [edge(ht_bf, src_sl[s], ea_sl[s], combo) for s in range(_SL)]
    return [_sc_scatter(ms[s], dst_sl[s], zF) for s in range(_SL)]


def kernel(x, edge_attr, edge_index,
           lin1_w, lin1_b, root1, bias1,
           lin2_w, lin2_b, root2, bias2,
           lin3_w, lin3_b, root3, bias3):
    src = edge_index[0].astype(jnp.int32).reshape(1, E)
    dst = edge_index[1].astype(jnp.int32).reshape(_SL, _NW, _JBLK, 128)
    src_sl = [src[:, s * _ES:(s + 1) * _ES] for s in range(_SL)]
    dst_sl = [dst[s] for s in range(_SL)]
    eat = jnp.zeros((16, E), jnp.float32)
    eat = eat.at[:DE, :].set(edge_attr.T)
    eat = eat.at[DE, :].set(1.0)
    eat = eat.astype(jnp.bfloat16)
    ea_sl = [eat[:, s * _ES:(s + 1) * _ES] for s in range(_SL)]

    zF = jnp.zeros((N, F), jnp.float32)
    xt_pad = jnp.pad(x, ((0, 0), (0, F - x.shape[1]))).T

    def combo_mat(ic, oc, lin_w, lin_b):
        K = ic * oc
        m = jnp.zeros((K, 16), jnp.float32)
        m = m.at[:, :DE].set(lin_w.T)
        m = m.at[:, DE].set(lin_b)
        return m.astype(jnp.bfloat16)

    cp = _layer(xt_pad, src_sl, ea_sl, dst_sl, zF, 32, 64, True,
                combo_mat(32, 64, lin1_w, lin1_b))
    ht = _make_finalize(32, 64)(*cp, *cp, xt_pad, root1, bias1.reshape(1, -1))

    p = _layer(ht, src_sl, ea_sl, dst_sl, zF, 64, 64, False,
               combo_mat(64, 64, lin2_w, lin2_b))
    ht = _make_finalize(64, 64)(*p, *cp, ht, root2, bias2.reshape(1, -1))

    p = _layer(ht, src_sl, ea_sl, dst_sl, zF, 64, 16, False,
               combo_mat(64, 16, lin3_w, lin3_b))
    ht = _make_finalize(64, 16)(*p, *cp, ht, root3, bias3.reshape(1, -1))

    hst = ht[:16, :]
    return _make_cbt(16)(hst, hst)


# speedup vs baseline: 1.3728x; 1.3728x over previous
import functools

import jax
import jax.numpy as jnp
from jax import lax
from jax.experimental import pallas as pl
from jax.experimental.pallas import tpu as pltpu
from jax.experimental.pallas import tpu_sc as plsc

N = 1024
E = 32768
DE = 8
F = 128

_NC = 2
_NS = 16
_NW = _NC * _NS
_SL = 2
_ES = E // _SL
_CHUNK = _ES // _NW
_JBLK = _CHUNK // 128


def _sc_mesh():
    return plsc.VectorSubcoreMesh(core_axis_name="c", subcore_axis_name="s")


@functools.partial(
    pl.kernel,
    mesh=_sc_mesh(),
    out_type=jax.ShapeDtypeStruct((_ES, F), jnp.float32),
    scratch_types=[
        pltpu.VMEM((_JBLK, 128), jnp.int32),
        pltpu.VMEM((_JBLK, 128, F), jnp.float32),
        pltpu.SemaphoreType.DMA,
        pltpu.SemaphoreType.DMA,
    ],
)
def _sc_gather(table_hbm, src_hbm, out_hbm, idx_v, rows_v, gsem, wsem):
    wid = lax.axis_index("s") * _NC + lax.axis_index("c")
    pltpu.sync_copy(src_hbm.at[wid], idx_v)
    gs = [
        pltpu.async_copy(table_hbm.at[idx_v.at[j]], rows_v.at[j], gsem)
        for j in range(_JBLK)
    ]
    wbs = []
    for j in range(_JBLK):
        gs[j].wait()
        wbs.append(
            pltpu.async_copy(
                rows_v.at[j],
                out_hbm.at[pl.ds(wid * _CHUNK + j * 128, 128)],
                wsem,
            )
        )
    for wb in wbs:
        wb.wait()


@functools.partial(
    pl.kernel,
    mesh=_sc_mesh(),
    out_type=jax.ShapeDtypeStruct((_NC, N, F), jnp.float32),
    scratch_types=[
        pltpu.VMEM((_JBLK, 128), jnp.int32),
        pltpu.VMEM((_JBLK, 128, F), jnp.float32),
        pltpu.VMEM_SHARED((N, F), jnp.float32),
        pltpu.SemaphoreType.DMA,
        pltpu.SemaphoreType.DMA,
    ],
)
def _sc_scatter(msg_hbm, dst_hbm, z_hbm, out_hbm, idx_v, msg_v, acc_sh,
                lsem, ssem):
    cid = lax.axis_index("c")
    sid = lax.axis_index("s")
    wid = sid * _NC + cid
    pltpu.sync_copy(dst_hbm.at[wid], idx_v)
    lds = [
        pltpu.async_copy(
            msg_hbm.at[pl.ds(wid * _CHUNK + j * 128, 128)],
            msg_v.at[j],
            lsem,
        )
        for j in range(_JBLK)
    ]

    @pl.when(sid == 0)
    def _():
        pltpu.sync_copy(z_hbm, acc_sh)

    plsc.subcore_barrier()
    scs = []
    for j in range(_JBLK):
        lds[j].wait()
        scs.append(
            pltpu.async_copy(
                msg_v.at[j], acc_sh.at[idx_v.at[j]], ssem, add=True
            )
        )
    for sc in scs:
        sc.wait()
    plsc.subcore_barrier()

    @pl.when(sid == 0)
    def _():
        pltpu.sync_copy(acc_sh, out_hbm.at[cid])


_EB = 4096


def _make_edge(ic, oc, with_ones):
    K = ic * oc

    def body(xs_ref, eat_ref, mt_ref, out_ref):
        wT = lax.dot_general(
            mt_ref[...], eat_ref[...], (((1,), (0,)), ((), ())),
            preferred_element_type=jnp.float32,
        )
        xsT = lax.transpose(xs_ref[...], (1, 0))
        acc = xsT[0:1, :] * jnp.maximum(wT[0:oc, :], 0.0)
        for i in range(1, ic):
            acc = acc + xsT[i:i + 1, :] * jnp.maximum(
                wT[i * oc:(i + 1) * oc, :], 0.0)
        msg = lax.transpose(acc, (1, 0))
        pad = jnp.zeros((_EB, F - oc - 1), jnp.float32)
        marker = jnp.full((_EB, 1), 1.0 if with_ones else 0.0, jnp.float32)
        out_ref[...] = jnp.concatenate([msg, marker, pad], axis=1)

    return pl.pallas_call(
        body,
        grid=(_ES // _EB,),
        in_specs=[
            pl.BlockSpec((_EB, F), lambda e: (e, 0)),
            pl.BlockSpec((16, _EB), lambda e: (0, e)),
            pl.BlockSpec((K, 16), lambda e: (0, 0)),
        ],
        out_specs=pl.BlockSpec((_EB, F), lambda e: (e, 0)),
        out_shape=jax.ShapeDtypeStruct((_ES, F), jnp.float32),
    )


def _make_finalize(ic, oc):
    BN = 128

    def body(*refs):
        p_refs = refs[0:_SL]
        c_refs = refs[_SL:2 * _SL]
        x_ref, r_ref, b_ref, out_ref = refs[2 * _SL:]
        s = p_refs[0][0] + p_refs[0][1]
        for pr in p_refs[1:]:
            s = s + pr[0] + pr[1]
        cnt = c_refs[0][0][:, 64:65] + c_refs[0][1][:, 64:65]
        for cr in c_refs[1:]:
            cnt = cnt + cr[0][:, 64:65] + cr[1][:, 64:65]
        inv = 1.0 / jnp.maximum(cnt, 1.0)
        xr = lax.dot_general(
            x_ref[:, 0:ic], r_ref[...], (((1,), (0,)), ((), ())),
            preferred_element_type=jnp.float32,
        )
        h = jnp.maximum(s[:, 0:oc] * inv + xr + b_ref[...], 0.0)
        pad = jnp.zeros((BN, F - oc), jnp.float32)
        out_ref[...] = jnp.concatenate([h, pad], axis=1)

    part_spec = pl.BlockSpec((_NC, BN, F), lambda i: (0, i, 0))
    return pl.pallas_call(
        body,
        grid=(N // BN,),
        in_specs=(
            [part_spec] * _SL
            + [part_spec] * _SL
            + [
                pl.BlockSpec((BN, F), lambda i: (i, 0)),
                pl.BlockSpec((ic, oc), lambda i: (0, 0)),
                pl.BlockSpec((1, oc), lambda i: (0, 0)),
            ]
        ),
        out_specs=pl.BlockSpec((BN, F), lambda i: (i, 0)),
        out_shape=jax.ShapeDtypeStruct((N, F), jnp.float32),
    )


def _make_cbt(dh):
    BT = 128

    def body(h_ref, ht_ref, out_ref):
        hi = h_ref[...]
        ht = ht_ref[...]
        acc = jnp.abs(hi[:, 0:1] - ht[0:1, :])
        for kk in range(1, dh):
            acc = acc + jnp.abs(hi[:, kk:kk + 1] - ht[kk:kk + 1, :])
        out_ref[...] = acc

    return pl.pallas_call(
        body,
        grid=(N // BT, N // BT),
        in_specs=[
            pl.BlockSpec((BT, dh), lambda i, j: (i, 0)),
            pl.BlockSpec((dh, BT), lambda i, j: (0, j)),
        ],
        out_specs=pl.BlockSpec((BT, BT), lambda i, j: (i, j)),
        out_shape=jax.ShapeDtypeStruct((N, N), jnp.float32),
    )


def _layer(h_tab, ea_sl, src_sl, dst_sl, zF, ic, oc, with_ones, combo):
    edge = _make_edge(ic, oc, with_ones)
    xs = [_sc_gather(h_tab, src_sl[s]) for s in range(_SL)]
    ms = [edge(xs[s], ea_sl[s], combo) for s in range(_SL)]
    return [_sc_scatter(ms[s], dst_sl[s], zF) for s in range(_SL)]


def kernel(x, edge_attr, edge_index,
           lin1_w, lin1_b, root1, bias1,
           lin2_w, lin2_b, root2, bias2,
           lin3_w, lin3_b, root3, bias3):
    src = edge_index[0].astype(jnp.int32).reshape(_SL, _NW, _JBLK, 128)
    dst = edge_index[1].astype(jnp.int32).reshape(_SL, _NW, _JBLK, 128)
    src_sl = [src[s] for s in range(_SL)]
    dst_sl = [dst[s] for s in range(_SL)]
    eat = jnp.zeros((16, E), jnp.float32)
    eat = eat.at[:DE, :].set(edge_attr.T)
    eat = eat.at[DE, :].set(1.0)
    eat = eat.astype(jnp.bfloat16)
    ea_sl = [eat[:, s * _ES:(s + 1) * _ES] for s in range(_SL)]

    zF = jnp.zeros((N, F), jnp.float32)
    x_pad = jnp.pad(x, ((0, 0), (0, F - x.shape[1])))

    def combo_mat(ic, oc, lin_w, lin_b):
        K = ic * oc
        m = jnp.zeros((K, 16), jnp.float32)
        m = m.at[:, :DE].set(lin_w.T)
        m = m.at[:, DE].set(lin_b)
        return m.astype(jnp.bfloat16)

    cp = _layer(x_pad, ea_sl, src_sl, dst_sl, zF, 32, 64, True,
                combo_mat(32, 64, lin1_w, lin1_b))
    h = _make_finalize(32, 64)(*cp, *cp, x_pad, root1, bias1.reshape(1, -1))

    p = _layer(h, ea_sl, src_sl, dst_sl, zF, 64, 64, False,
               combo_mat(64, 64, lin2_w, lin2_b))
    h = _make_finalize(64, 64)(*p, *cp, h, root2, bias2.reshape(1, -1))

    p = _layer(h, ea_sl, src_sl, dst_sl, zF, 64, 16, False,
               combo_mat(64, 16, lin3_w, lin3_b))
    h = _make_finalize(64, 16)(*p, *cp, h, root3, bias3.reshape(1, -1))

    hs = h[:, :16]
    return _make_cbt(16)(hs, hs.T)
